# baseline (speedup 1.0000x reference)
import functools

import jax
import jax.numpy as jnp
from jax import lax
from jax.experimental import pallas as pl
from jax.experimental.pallas import tpu as pltpu
from jax.experimental.pallas import tpu_sc as plsc

B = 4
SEQ = 8192
D = 1024
NC = 2
NS = 16
NW = NC * NS
ROWS_W = SEQ // NW
R = 16
CHUNKS = ROWS_W // R
SLICES = R * D // 16
STEPS = CHUNKS * B
NBX = 4
NBP = 2

_mesh = plsc.VectorSubcoreMesh(core_axis_name="c", subcore_axis_name="s")


@functools.partial(
    pl.kernel,
    out_type=jax.ShapeDtypeStruct((B, SEQ, D), jnp.float32),
    mesh=_mesh,
    scratch_types=[
        [pltpu.VMEM((R, D), jnp.float32) for _ in range(NBX)],
        [pltpu.VMEM((R, D), jnp.float32) for _ in range(NBP)],
        [pltpu.SemaphoreType.DMA for _ in range(NBX)],
        [pltpu.SemaphoreType.DMA for _ in range(NBP)],
        [pltpu.SemaphoreType.DMA for _ in range(NBX)],
    ],
)
def _sc_add(x_hbm, pos_hbm, out_hbm, x_v, pos_v, sem_x, sem_p, sem_o):
    wid = lax.axis_index("s") * NC + lax.axis_index("c")
    row0 = wid * ROWS_W

    def start_pos(c):
        r = row0 + c * R
        return pltpu.async_copy(pos_hbm.at[pl.ds(r, R), :],
                                pos_v[c % NBP], sem_p[c % NBP])

    def start_in(t):
        c, b = divmod(t, B)
        r = row0 + c * R
        return pltpu.async_copy(x_hbm.at[b, pl.ds(r, R), :],
                                x_v[t % NBX], sem_x[t % NBX])

    def start_out(t):
        c, b = divmod(t, B)
        r = row0 + c * R
        return pltpu.async_copy(x_v[t % NBX],
                                out_hbm.at[b, pl.ds(r, R), :],
                                sem_o[t % NBX])

    pos_h = {0: start_pos(0), 1: start_pos(1)}
    in_h = {0: start_in(0), 1: start_in(1)}
    out_h = {}
    for t in range(STEPS):
        c, b = divmod(t, B)
        if t + 2 < STEPS:
            if t - 2 in out_h:
                out_h.pop(t - 2).wait()
            in_h[t + 2] = start_in(t + 2)
        in_h.pop(t).wait()
        if b == 0:
            pos_h.pop(c).wait()

        @plsc.parallel_loop(0, SLICES, 1, unroll=8)
        def add_b(i, xb=x_v[t % NBX], pv=pos_v[c % NBP]):
            r = i >> 6
            col = (i & 63) * 16
            plsc.addupdate(xb.at[r, pl.ds(col, 16)], pv[r, pl.ds(col, 16)])

        out_h[t] = start_out(t)
        if b == B - 1 and c + 2 < CHUNKS:
            pos_h[c + 2] = start_pos(c + 2)
    for t in sorted(out_h):
        out_h[t].wait()


def kernel(x, pos_table):
    return _sc_add(x, pos_table)


# speedup vs baseline: 1.0059x; 1.0059x over previous
import functools

import jax
import jax.numpy as jnp
from jax import lax
from jax.experimental import pallas as pl
from jax.experimental.pallas import tpu as pltpu
from jax.experimental.pallas import tpu_sc as plsc

B = 4
SEQ = 8192
D = 1024
NC = 2
NS = 16
NW = NC * NS
ROWS_W = SEQ // NW
R = 16
CHUNKS = ROWS_W // R
SLICES = R * D // 16
STEPS = CHUNKS * B
NBX = 5
NBP = 2
AHEAD = NBX - 2

_mesh = plsc.VectorSubcoreMesh(core_axis_name="c", subcore_axis_name="s")


@functools.partial(
    pl.kernel,
    out_type=jax.ShapeDtypeStruct((B, SEQ, D), jnp.float32),
    mesh=_mesh,
    scratch_types=[
        [pltpu.VMEM((R, D), jnp.float32) for _ in range(NBX)],
        [pltpu.VMEM((R, D), jnp.float32) for _ in range(NBP)],
        [pltpu.SemaphoreType.DMA for _ in range(NBX)],
        [pltpu.SemaphoreType.DMA for _ in range(NBP)],
        [pltpu.SemaphoreType.DMA for _ in range(NBX)],
    ],
)
def _sc_add(x_hbm, pos_hbm, out_hbm, x_v, pos_v, sem_x, sem_p, sem_o):
    wid = lax.axis_index("s") * NC + lax.axis_index("c")
    row0 = wid * ROWS_W

    def start_pos(c):
        r = row0 + c * R
        return pltpu.async_copy(pos_hbm.at[pl.ds(r, R), :],
                                pos_v[c % NBP], sem_p[c % NBP])

    def start_in(t):
        c, b = divmod(t, B)
        r = row0 + c * R
        return pltpu.async_copy(x_hbm.at[b, pl.ds(r, R), :],
                                x_v[t % NBX], sem_x[t % NBX])

    def start_out(t):
        c, b = divmod(t, B)
        r = row0 + c * R
        return pltpu.async_copy(x_v[t % NBX],
                                out_hbm.at[b, pl.ds(r, R), :],
                                sem_o[t % NBX])

    pos_h = {0: start_pos(0), 1: start_pos(1)}
    in_h = {k: start_in(k) for k in range(AHEAD)}
    out_h = {}
    for t in range(STEPS):
        c, b = divmod(t, B)
        if t + AHEAD < STEPS:
            if t + AHEAD - NBX in out_h:
                out_h.pop(t + AHEAD - NBX).wait()
            in_h[t + AHEAD] = start_in(t + AHEAD)
        in_h.pop(t).wait()
        if b == 0:
            pos_h.pop(c).wait()

        @plsc.parallel_loop(0, SLICES, 1, unroll=8)
        def add_b(i, xb=x_v[t % NBX], pv=pos_v[c % NBP]):
            r = i >> 6
            col = (i & 63) * 16
            plsc.addupdate(xb.at[r, pl.ds(col, 16)], pv[r, pl.ds(col, 16)])

        out_h[t] = start_out(t)
        if b == B - 1 and c + 2 < CHUNKS:
            pos_h[c + 2] = start_pos(c + 2)
    for t in sorted(out_h):
        out_h[t].wait()


def kernel(x, pos_table):
    return _sc_add(x, pos_table)
